# baseline (speedup 1.0000x reference)
import functools

import jax
import jax.numpy as jnp
from jax import lax
from jax.experimental import pallas as pl
from jax.experimental.pallas import tpu as pltpu
from jax.experimental.pallas import tpu_sc as plsc

NC, NS, L = 2, 16, 16
NW = NC * NS


def kernel(x_sparse, x_dense, table, W_dense):
    B, F = x_sparse.shape
    _, V = table.shape
    _, D = x_dense.shape

    b_per_w = B // NW
    n_chunks = b_per_w // L

    xs_blk = (
        x_sparse.astype(jnp.int32).T.reshape(F, NW, b_per_w).transpose(1, 0, 2)
    )
    xd_blk = x_dense.T.reshape(D, NW, b_per_w).transpose(1, 0, 2)
    tab_flat = table.reshape(-1)
    w_blk = jnp.broadcast_to(W_dense, (D, L))

    mesh = plsc.VectorSubcoreMesh(
        core_axis_name="c", subcore_axis_name="s", num_cores=NC, num_subcores=NS
    )

    @functools.partial(
        pl.kernel,
        out_type=jax.ShapeDtypeStruct((B,), jnp.float32),
        mesh=mesh,
        scratch_types=[
            pltpu.VMEM((F, b_per_w), jnp.int32),
            pltpu.VMEM((F, b_per_w), jnp.int32),
            pltpu.VMEM((F, b_per_w), jnp.float32),
            pltpu.VMEM((D, b_per_w), jnp.float32),
            pltpu.VMEM((D, L), jnp.float32),
            pltpu.VMEM((b_per_w,), jnp.float32),
            pltpu.SemaphoreType.DMA,
        ],
    )
    def sc_kernel(xs_hbm, xd_hbm, tab_hbm, w_hbm, out_hbm, idx_v, idxf_v, vals_v, xd_v, w_v, acc_v, sem):
        wid = lax.axis_index("s") * NC + lax.axis_index("c")
        base = wid * b_per_w

        pltpu.sync_copy(xs_hbm.at[wid], idx_v)
        pltpu.sync_copy(xd_hbm.at[wid], xd_v)
        pltpu.sync_copy(w_hbm, w_v)

        for f in range(F):
            for c in range(n_chunks):
                sl = pl.ds(c * L, L)
                idxf_v[f, sl] = idx_v[f, sl] + f * V

        copies = [
            pltpu.async_copy(tab_hbm.at[idxf_v.at[f]], vals_v.at[f], sem)
            for f in range(F)
        ]

        w_bcast = [w_v[d, :] for d in range(D)]

        for cp in copies:
            cp.wait()

        for c in range(n_chunks):
            sl = pl.ds(c * L, L)
            acc = vals_v[0, sl]
            for f in range(1, F):
                acc = acc + vals_v[f, sl]
            for d in range(D):
                acc = acc + xd_v[d, sl] * w_bcast[d]
            acc_v[sl] = acc

        pltpu.sync_copy(acc_v, out_hbm.at[pl.ds(base, b_per_w)])

    out = sc_kernel(xs_blk, xd_blk, tab_flat, w_blk)
    return out.reshape(B, 1)
